# baseline (speedup 1.0000x reference)
import functools

import jax
import jax.numpy as jnp
from jax import lax
from jax.experimental import pallas as pl
from jax.experimental.pallas import tpu as pltpu
from jax.experimental.pallas import tpu_sc as plsc

N_NODES = 10000
D = 128
NC = 2
NS = 16
NW = NC * NS
CH = 64
S = 160
E_PAD = NW * S * CH
N_PAD = N_NODES + 112
SUB = N_PAD // NS
L = 16

_f32 = jnp.float32


def _fill_2d(ref, nrows, ncols, value):
    v = jnp.full((L,), value, _f32)

    def row(r, carry):
        for c in range(ncols // L):
            ref[r, pl.ds(c * L, L)] = v
        return carry

    lax.fori_loop(0, nrows, row, 0)


def _fill_1d(ref, n, value):
    v = jnp.full((L,), value, _f32)

    def step(i, carry):
        ref[pl.ds(i * L, L)] = v
        return carry

    lax.fori_loop(0, n // L, step, 0)


def _copy_chunks(src_ref, dst_ref, base):
    for off in range(0, SUB, CH):
        sz = min(CH, SUB - off)
        pltpu.sync_copy(src_ref.at[pl.ds(0, sz)], dst_ref.at[pl.ds(base + off, sz)])


def _make_sc_agg(with_cnt):
    mesh = plsc.VectorSubcoreMesh(core_axis_name="c", subcore_axis_name="s",
                                  num_cores=NC, num_subcores=NS)
    out_type = [jax.ShapeDtypeStruct((NC, N_PAD, D), _f32)]
    if with_cnt:
        out_type.append(jax.ShapeDtypeStruct((NW, N_PAD), _f32))
    NI = 8
    NR = 4
    scratch = (
        [pltpu.VMEM((2, CH), jnp.int32) for _ in range(NI)]
        + [pltpu.VMEM((CH, D), _f32) for _ in range(NR)]
        + [pltpu.VMEM_SHARED((N_PAD, D), _f32)]
        + [pltpu.SemaphoreType.DMA for _ in range(NR)]
        + [pltpu.SemaphoreType.DMA,
           pltpu.SemaphoreType.DMA]
    )
    if with_cnt:
        scratch.append(pltpu.VMEM((N_PAD,), _f32))

    def body(*refs):
        f_hbm, edges, psum = refs[:3]
        refs = refs[3:]
        if with_cnt:
            pcnt, refs = refs[0], refs[1:]
            cnt_v = refs[-1]
            refs = refs[:-1]
        idx = refs[:NI]
        rows = refs[NI:NI + NR]
        acc_sh = refs[NI + NR]
        gsem = refs[NI + NR + 1:NI + NR + 1 + NR]
        isem, ssem = refs[NI + NR + 1 + NR:NI + NR + 3 + NR]
        c = lax.axis_index("c")
        s = lax.axis_index("s")
        w = c * NS + s

        _fill_2d(rows[0], CH, D, 0.0)
        _copy_chunks(rows[0], acc_sh, s * SUB)
        _fill_2d(rows[NR - 1], CH, D, 0.0)
        if with_cnt:
            _fill_1d(cnt_v, N_PAD, 0.0)
        plsc.subcore_barrier()

        ones16 = jnp.ones((L,), _f32)

        def step(j, a, ci, ia):
            i3 = (ia + 3) % NI
            i4 = (ia + 4) % NI
            pltpu.make_async_copy(edges.at[w, 0], idx[i3], isem).wait()
            pltpu.make_async_copy(rows[ci], acc_sh.at[idx[0].at[1]], ssem).wait()
            pltpu.async_copy(f_hbm.at[idx[i3].at[0]], rows[ci], gsem[ci])
            pltpu.make_async_copy(f_hbm.at[idx[0].at[0]], rows[a], gsem[a]).wait()
            pltpu.async_copy(rows[a], acc_sh.at[idx[ia].at[1]], ssem, add=True)
            if with_cnt:
                for k in range(CH // L):
                    iv = idx[ia][1, pl.ds(k * L, L)]
                    plsc.addupdate_scatter(cnt_v, [iv], ones16)
            pltpu.async_copy(edges.at[w, j + 4], idx[i4], isem)

        for i in range(NR - 1):
            pltpu.async_copy(edges.at[w, i], idx[i], isem)
            pltpu.make_async_copy(edges.at[w, i], idx[i], isem).wait()
            pltpu.async_copy(f_hbm.at[idx[i].at[0]], rows[i], gsem[i])
        pltpu.async_copy(rows[NR - 1], acc_sh.at[idx[0].at[1]], ssem, add=True)
        pltpu.async_copy(edges.at[w, NR - 1], idx[NR - 1], isem)

        def eight(t, carry):
            j = t * NI
            for u in range(NI):
                step(j + u, u % NR, (u + 3) % NR, u)
            return carry

        lax.fori_loop(0, S // NI, eight, 0)
        pltpu.make_async_copy(rows[0], acc_sh.at[idx[0].at[1]], ssem).wait()
        for i in range(NR - 1):
            pltpu.make_async_copy(f_hbm.at[idx[0].at[0]], rows[(S + i) % NR],
                                  gsem[(S + i) % NR]).wait()
        pltpu.make_async_copy(edges.at[w, 0], idx[0], isem).wait()
        plsc.subcore_barrier()

        pltpu.sync_copy(acc_sh.at[pl.ds(s * SUB, SUB)],
                        psum.at[c, pl.ds(s * SUB, SUB)])
        if with_cnt:
            pltpu.sync_copy(cnt_v, pcnt.at[w])

    return pl.kernel(
        body, out_type=out_type, mesh=mesh, scratch_types=scratch,
        compiler_params=pltpu.CompilerParams(needs_layout_passes=False))


@functools.cache
def _sc_agg(with_cnt):
    return _make_sc_agg(with_cnt)


def _dense_body(do_relu, p_ref, pc_ref, x_ref, wl_ref, wr_ref, b_ref, o_ref):
    acc = p_ref[0] + p_ref[1]
    cnt = lax.dot_general(pc_ref[...], jnp.ones((NW, 1), _f32),
                          (((0,), (0,)), ((), ())), preferred_element_type=_f32)
    agg = acc * (1.0 / jnp.maximum(cnt, 1.0))
    y = lax.dot_general(agg, wl_ref[...], (((1,), (1,)), ((), ())),
                        preferred_element_type=_f32)
    y = y + lax.dot_general(x_ref[...], wr_ref[...], (((1,), (1,)), ((), ())),
                            preferred_element_type=_f32)
    y = y + b_ref[...]
    if do_relu:
        y = jnp.maximum(y, 0.0)
    o_ref[...] = y


def _dense(p, pc, x, Wl, Wr, b, do_relu):
    R = 1024
    G = -(-N_NODES // R)
    return pl.pallas_call(
        functools.partial(_dense_body, do_relu),
        grid=(G,),
        in_specs=[
            pl.BlockSpec((NC, R, D), lambda i: (0, i, 0)),
            pl.BlockSpec((NW, R), lambda i: (0, i)),
            pl.BlockSpec((R, D), lambda i: (i, 0)),
            pl.BlockSpec((D, D), lambda i: (0, 0)),
            pl.BlockSpec((D, D), lambda i: (0, 0)),
            pl.BlockSpec((1, D), lambda i: (0, 0)),
        ],
        out_specs=pl.BlockSpec((R, D), lambda i: (i, 0)),
        out_shape=jax.ShapeDtypeStruct((N_NODES, D), _f32),
    )(p, pc, x, Wl, Wr, b)


def kernel(x, edge_index, Wl1, Wr1, b1, Wl2, Wr2, b2):
    src = edge_index[0].astype(jnp.int32)
    dst = edge_index[1].astype(jnp.int32)
    e = src.shape[0]
    pad = E_PAD - e
    pad_src = (jnp.arange(pad, dtype=jnp.int32) * 97) % N_NODES
    pad_dst = N_NODES + (jnp.arange(pad, dtype=jnp.int32) % 16)
    srcs = jnp.concatenate([src, pad_src]).reshape(NW, S, CH)
    dsts = jnp.concatenate([dst, pad_dst]).reshape(NW, S, CH)
    edges = jnp.pad(jnp.stack([srcs, dsts], axis=2),
                    ((0, 0), (0, 4), (0, 0), (0, 0)), mode="wrap")
    b1r = b1.reshape(1, D)
    b2r = b2.reshape(1, D)

    p1, c1 = _sc_agg(True)(x, edges)
    h = _dense(p1, c1, x, Wl1, Wr1, b1r, True)
    (p2,) = _sc_agg(False)(h, edges)
    out = _dense(p2, c1, h, Wl2, Wr2, b2r, False)
    return out


# speedup vs baseline: 1.0782x; 1.0782x over previous
import functools

import jax
import jax.numpy as jnp
from jax import lax
from jax.experimental import pallas as pl
from jax.experimental.pallas import tpu as pltpu
from jax.experimental.pallas import tpu_sc as plsc

N_NODES = 10000
D = 128
NC = 2
NS = 16
NW = NC * NS
CH = 80
S = 126
E_PAD = NW * S * CH
N_PAD = N_NODES + 112
SUB = N_PAD // NS
L = 16

_f32 = jnp.float32


def _fill_2d(ref, nrows, ncols, value):
    v = jnp.full((L,), value, _f32)

    def row(r, carry):
        for c in range(ncols // L):
            ref[r, pl.ds(c * L, L)] = v
        return carry

    lax.fori_loop(0, nrows, row, 0)


def _fill_1d(ref, n, value):
    v = jnp.full((L,), value, _f32)

    def step(i, carry):
        ref[pl.ds(i * L, L)] = v
        return carry

    lax.fori_loop(0, n // L, step, 0)


def _copy_chunks(src_ref, dst_ref, base):
    for off in range(0, SUB, CH):
        sz = min(CH, SUB - off)
        pltpu.sync_copy(src_ref.at[pl.ds(0, sz)], dst_ref.at[pl.ds(base + off, sz)])


def _make_sc_agg(with_cnt):
    mesh = plsc.VectorSubcoreMesh(core_axis_name="c", subcore_axis_name="s",
                                  num_cores=NC, num_subcores=NS)
    out_type = [jax.ShapeDtypeStruct((NC, N_PAD, D), _f32)]
    if with_cnt:
        out_type.append(jax.ShapeDtypeStruct((NW, N_PAD), _f32))
    NI = 6
    NR = 3
    scratch = (
        [pltpu.VMEM((2, CH), jnp.int32) for _ in range(NI)]
        + [pltpu.VMEM((CH, D), _f32) for _ in range(NR)]
        + [pltpu.VMEM_SHARED((N_PAD, D), _f32)]
        + [pltpu.SemaphoreType.DMA for _ in range(NR)]
        + [pltpu.SemaphoreType.DMA,
           pltpu.SemaphoreType.DMA]
    )
    if with_cnt:
        scratch.append(pltpu.VMEM((N_PAD,), _f32))

    def body(*refs):
        f_hbm, edges, psum = refs[:3]
        refs = refs[3:]
        if with_cnt:
            pcnt, refs = refs[0], refs[1:]
            cnt_v = refs[-1]
            refs = refs[:-1]
        idx = refs[:NI]
        rows = refs[NI:NI + NR]
        acc_sh = refs[NI + NR]
        gsem = refs[NI + NR + 1:NI + NR + 1 + NR]
        isem, ssem = refs[NI + NR + 1 + NR:NI + NR + 3 + NR]
        c = lax.axis_index("c")
        s = lax.axis_index("s")
        w = c * NS + s

        _fill_2d(rows[0], CH, D, 0.0)
        _copy_chunks(rows[0], acc_sh, s * SUB)
        _fill_2d(rows[NR - 1], CH, D, 0.0)
        if with_cnt:
            _fill_1d(cnt_v, N_PAD, 0.0)
        plsc.subcore_barrier()

        ones16 = jnp.ones((L,), _f32)

        def step(j, a, ci, ia):
            ig = (ia + NR - 1) % NI
            inx = (ia + NR) % NI
            pltpu.make_async_copy(edges.at[w, 0], idx[ig], isem).wait()
            pltpu.make_async_copy(rows[ci], acc_sh.at[idx[0].at[1]], ssem).wait()
            pltpu.async_copy(f_hbm.at[idx[ig].at[0]], rows[ci], gsem[ci])
            pltpu.make_async_copy(f_hbm.at[idx[0].at[0]], rows[a], gsem[a]).wait()
            pltpu.async_copy(rows[a], acc_sh.at[idx[ia].at[1]], ssem, add=True)
            if with_cnt:
                for k in range(CH // L):
                    iv = idx[ia][1, pl.ds(k * L, L)]
                    plsc.addupdate_scatter(cnt_v, [iv], ones16)
            pltpu.async_copy(edges.at[w, j + NR], idx[inx], isem)

        for i in range(NR - 1):
            pltpu.async_copy(edges.at[w, i], idx[i], isem)
            pltpu.make_async_copy(edges.at[w, i], idx[i], isem).wait()
            pltpu.async_copy(f_hbm.at[idx[i].at[0]], rows[i], gsem[i])
        pltpu.async_copy(rows[NR - 1], acc_sh.at[idx[0].at[1]], ssem, add=True)
        pltpu.async_copy(edges.at[w, NR - 1], idx[NR - 1], isem)

        def unrolled(t, carry):
            j = t * NI
            for u in range(NI):
                step(j + u, u % NR, (u + NR - 1) % NR, u)
            return carry

        lax.fori_loop(0, S // NI, unrolled, 0)
        pltpu.make_async_copy(rows[0], acc_sh.at[idx[0].at[1]], ssem).wait()
        for i in range(NR - 1):
            pltpu.make_async_copy(f_hbm.at[idx[0].at[0]], rows[(S + i) % NR],
                                  gsem[(S + i) % NR]).wait()
        pltpu.make_async_copy(edges.at[w, 0], idx[0], isem).wait()
        plsc.subcore_barrier()

        pltpu.sync_copy(acc_sh.at[pl.ds(s * SUB, SUB)],
                        psum.at[c, pl.ds(s * SUB, SUB)])
        if with_cnt:
            pltpu.sync_copy(cnt_v, pcnt.at[w])

    return pl.kernel(
        body, out_type=out_type, mesh=mesh, scratch_types=scratch,
        compiler_params=pltpu.CompilerParams(needs_layout_passes=False))


@functools.cache
def _sc_agg(with_cnt):
    return _make_sc_agg(with_cnt)


def _dense_body(do_relu, p_ref, pc_ref, x_ref, wl_ref, wr_ref, b_ref, o_ref):
    acc = p_ref[0] + p_ref[1]
    cnt = lax.dot_general(pc_ref[...], jnp.ones((NW, 1), _f32),
                          (((0,), (0,)), ((), ())), preferred_element_type=_f32)
    agg = acc * (1.0 / jnp.maximum(cnt, 1.0))
    y = lax.dot_general(agg, wl_ref[...], (((1,), (1,)), ((), ())),
                        preferred_element_type=_f32)
    y = y + lax.dot_general(x_ref[...], wr_ref[...], (((1,), (1,)), ((), ())),
                            preferred_element_type=_f32)
    y = y + b_ref[...]
    if do_relu:
        y = jnp.maximum(y, 0.0)
    o_ref[...] = y


def _dense(p, pc, x, Wl, Wr, b, do_relu):
    R = 1024
    G = -(-N_NODES // R)
    return pl.pallas_call(
        functools.partial(_dense_body, do_relu),
        grid=(G,),
        in_specs=[
            pl.BlockSpec((NC, R, D), lambda i: (0, i, 0)),
            pl.BlockSpec((NW, R), lambda i: (0, i)),
            pl.BlockSpec((R, D), lambda i: (i, 0)),
            pl.BlockSpec((D, D), lambda i: (0, 0)),
            pl.BlockSpec((D, D), lambda i: (0, 0)),
            pl.BlockSpec((1, D), lambda i: (0, 0)),
        ],
        out_specs=pl.BlockSpec((R, D), lambda i: (i, 0)),
        out_shape=jax.ShapeDtypeStruct((N_NODES, D), _f32),
    )(p, pc, x, Wl, Wr, b)


def kernel(x, edge_index, Wl1, Wr1, b1, Wl2, Wr2, b2):
    src = edge_index[0].astype(jnp.int32)
    dst = edge_index[1].astype(jnp.int32)
    e = src.shape[0]
    pad = E_PAD - e
    pad_src = (jnp.arange(pad, dtype=jnp.int32) * 97) % N_NODES
    pad_dst = N_NODES + (jnp.arange(pad, dtype=jnp.int32) % 16)
    srcs = jnp.concatenate([src, pad_src]).reshape(NW, S, CH)
    dsts = jnp.concatenate([dst, pad_dst]).reshape(NW, S, CH)
    edges = jnp.pad(jnp.stack([srcs, dsts], axis=2),
                    ((0, 0), (0, 4), (0, 0), (0, 0)), mode="wrap")
    b1r = b1.reshape(1, D)
    b2r = b2.reshape(1, D)

    p1, c1 = _sc_agg(True)(x, edges)
    h = _dense(p1, c1, x, Wl1, Wr1, b1r, True)
    (p2,) = _sc_agg(False)(h, edges)
    out = _dense(p2, c1, h, Wl2, Wr2, b2r, False)
    return out


# speedup vs baseline: 1.1076x; 1.0273x over previous
import functools

import jax
import jax.numpy as jnp
from jax import lax
from jax.experimental import pallas as pl
from jax.experimental.pallas import tpu as pltpu
from jax.experimental.pallas import tpu_sc as plsc

N_NODES = 10000
D = 128
NC = 2
NS = 16
NW = NC * NS
CH = 88
S = 114
E_PAD = NW * S * CH
N_PAD = N_NODES + 112
SUB = N_PAD // NS
L = 16

_f32 = jnp.float32


def _fill_2d(ref, nrows, ncols, value):
    v = jnp.full((L,), value, _f32)

    def row(r, carry):
        for c in range(ncols // L):
            ref[r, pl.ds(c * L, L)] = v
        return carry

    lax.fori_loop(0, nrows, row, 0)


def _fill_1d(ref, n, value):
    v = jnp.full((L,), value, _f32)

    def step(i, carry):
        ref[pl.ds(i * L, L)] = v
        return carry

    lax.fori_loop(0, n // L, step, 0)


def _copy_chunks(src_ref, dst_ref, base):
    for off in range(0, SUB, CH):
        sz = min(CH, SUB - off)
        pltpu.sync_copy(src_ref.at[pl.ds(0, sz)], dst_ref.at[pl.ds(base + off, sz)])


def _make_sc_agg(with_cnt):
    mesh = plsc.VectorSubcoreMesh(core_axis_name="c", subcore_axis_name="s",
                                  num_cores=NC, num_subcores=NS)
    out_type = [jax.ShapeDtypeStruct((NC, N_PAD, D), _f32)]
    if with_cnt:
        out_type.append(jax.ShapeDtypeStruct((NW, N_PAD), _f32))
    NI = 6
    NR = 3
    scratch = (
        [pltpu.VMEM((2, CH), jnp.int32) for _ in range(NI)]
        + [pltpu.VMEM((CH, D), _f32) for _ in range(NR)]
        + [pltpu.VMEM_SHARED((N_PAD, D), _f32)]
        + [pltpu.SemaphoreType.DMA for _ in range(NR)]
        + [pltpu.SemaphoreType.DMA,
           pltpu.SemaphoreType.DMA]
    )
    if with_cnt:
        scratch.append(pltpu.VMEM((N_PAD,), _f32))

    def body(*refs):
        f_hbm, edges, psum = refs[:3]
        refs = refs[3:]
        if with_cnt:
            pcnt, refs = refs[0], refs[1:]
            cnt_v = refs[-1]
            refs = refs[:-1]
        idx = refs[:NI]
        rows = refs[NI:NI + NR]
        acc_sh = refs[NI + NR]
        gsem = refs[NI + NR + 1:NI + NR + 1 + NR]
        isem, ssem = refs[NI + NR + 1 + NR:NI + NR + 3 + NR]
        c = lax.axis_index("c")
        s = lax.axis_index("s")
        w = c * NS + s

        _fill_2d(rows[0], CH, D, 0.0)
        _copy_chunks(rows[0], acc_sh, s * SUB)
        _fill_2d(rows[NR - 1], CH, D, 0.0)
        if with_cnt:
            _fill_1d(cnt_v, N_PAD, 0.0)
        plsc.subcore_barrier()

        ones16 = jnp.ones((L,), _f32)

        def step(j, a, ci, ia):
            ig = (ia + NR - 1) % NI
            inx = (ia + NR) % NI
            pltpu.make_async_copy(edges.at[w, 0], idx[ig], isem).wait()
            pltpu.make_async_copy(rows[ci], acc_sh.at[idx[0].at[1]], ssem).wait()
            pltpu.async_copy(f_hbm.at[idx[ig].at[0]], rows[ci], gsem[ci])
            pltpu.make_async_copy(f_hbm.at[idx[0].at[0]], rows[a], gsem[a]).wait()
            pltpu.async_copy(rows[a], acc_sh.at[idx[ia].at[1]], ssem, add=True)
            if with_cnt:
                for k in range(CH // L):
                    iv = idx[ia][1, pl.ds(k * L, L)]
                    plsc.addupdate_scatter(cnt_v, [iv], ones16)
            pltpu.async_copy(edges.at[w, j + NR], idx[inx], isem)

        for i in range(NR - 1):
            pltpu.async_copy(edges.at[w, i], idx[i], isem)
            pltpu.make_async_copy(edges.at[w, i], idx[i], isem).wait()
            pltpu.async_copy(f_hbm.at[idx[i].at[0]], rows[i], gsem[i])
        pltpu.async_copy(rows[NR - 1], acc_sh.at[idx[0].at[1]], ssem, add=True)
        pltpu.async_copy(edges.at[w, NR - 1], idx[NR - 1], isem)

        def unrolled(t, carry):
            j = t * NI
            for u in range(NI):
                step(j + u, u % NR, (u + NR - 1) % NR, u)
            return carry

        lax.fori_loop(0, S // NI, unrolled, 0)
        pltpu.make_async_copy(rows[0], acc_sh.at[idx[0].at[1]], ssem).wait()
        for i in range(NR - 1):
            pltpu.make_async_copy(f_hbm.at[idx[0].at[0]], rows[(S + i) % NR],
                                  gsem[(S + i) % NR]).wait()
        pltpu.make_async_copy(edges.at[w, 0], idx[0], isem).wait()
        plsc.subcore_barrier()

        pltpu.sync_copy(acc_sh.at[pl.ds(s * SUB, SUB)],
                        psum.at[c, pl.ds(s * SUB, SUB)])
        if with_cnt:
            pltpu.sync_copy(cnt_v, pcnt.at[w])

    return pl.kernel(
        body, out_type=out_type, mesh=mesh, scratch_types=scratch,
        compiler_params=pltpu.CompilerParams(needs_layout_passes=False))


@functools.cache
def _sc_agg(with_cnt):
    return _make_sc_agg(with_cnt)


def _dense_body(do_relu, p_ref, pc_ref, x_ref, wl_ref, wr_ref, b_ref, o_ref):
    acc = p_ref[0] + p_ref[1]
    cnt = lax.dot_general(pc_ref[...], jnp.ones((NW, 1), _f32),
                          (((0,), (0,)), ((), ())), preferred_element_type=_f32)
    agg = acc * (1.0 / jnp.maximum(cnt, 1.0))
    y = lax.dot_general(agg, wl_ref[...], (((1,), (1,)), ((), ())),
                        preferred_element_type=_f32)
    y = y + lax.dot_general(x_ref[...], wr_ref[...], (((1,), (1,)), ((), ())),
                            preferred_element_type=_f32)
    y = y + b_ref[...]
    if do_relu:
        y = jnp.maximum(y, 0.0)
    o_ref[...] = y


def _dense(p, pc, x, Wl, Wr, b, do_relu):
    R = 1024
    G = -(-N_NODES // R)
    return pl.pallas_call(
        functools.partial(_dense_body, do_relu),
        grid=(G,),
        in_specs=[
            pl.BlockSpec((NC, R, D), lambda i: (0, i, 0)),
            pl.BlockSpec((NW, R), lambda i: (0, i)),
            pl.BlockSpec((R, D), lambda i: (i, 0)),
            pl.BlockSpec((D, D), lambda i: (0, 0)),
            pl.BlockSpec((D, D), lambda i: (0, 0)),
            pl.BlockSpec((1, D), lambda i: (0, 0)),
        ],
        out_specs=pl.BlockSpec((R, D), lambda i: (i, 0)),
        out_shape=jax.ShapeDtypeStruct((N_NODES, D), _f32),
    )(p, pc, x, Wl, Wr, b)


def kernel(x, edge_index, Wl1, Wr1, b1, Wl2, Wr2, b2):
    src = edge_index[0].astype(jnp.int32)
    dst = edge_index[1].astype(jnp.int32)
    e = src.shape[0]
    pad = E_PAD - e
    pad_src = (jnp.arange(pad, dtype=jnp.int32) * 97) % N_NODES
    pad_dst = N_NODES + (jnp.arange(pad, dtype=jnp.int32) % 16)
    srcs = jnp.concatenate([src, pad_src]).reshape(NW, S, CH)
    dsts = jnp.concatenate([dst, pad_dst]).reshape(NW, S, CH)
    edges = jnp.pad(jnp.stack([srcs, dsts], axis=2),
                    ((0, 0), (0, 4), (0, 0), (0, 0)), mode="wrap")
    b1r = b1.reshape(1, D)
    b2r = b2.reshape(1, D)

    p1, c1 = _sc_agg(True)(x, edges)
    h = _dense(p1, c1, x, Wl1, Wr1, b1r, True)
    (p2,) = _sc_agg(False)(h, edges)
    out = _dense(p2, c1, h, Wl2, Wr2, b2r, False)
    return out
